# baseline (speedup 1.0000x reference)
import functools

import jax
import jax.numpy as jnp
from jax import lax
from jax.experimental import pallas as pl
from jax.experimental.pallas import tpu as pltpu
from jax.experimental.pallas import tpu_sc as plsc

N = 4096
K = 20
G = 4
EPS = 1e-5
BLK = 512
NW = 32
NPW = N // NW
CN = 8


def _knn_and_mm(feat, bcol, bblk, Wd, Wbs, c):
    din = feat.shape[1]
    dout = Wd.shape[1]
    dpad = Wbs.shape[1]
    nblk = N // BLK

    CT = 512
    NT = N // CT

    def body(fb_ref, f_ref, bcol_ref, bcol3_ref, bblk_ref, wd_ref, wbs_ref,
             c_ref, idx_ref, as_ref, bs_ref, d_ref):
        i = pl.program_id(0)
        fb = fb_ref[...]
        bblk = bblk_ref[...]
        bcol = bcol_ref[...]
        row = lax.broadcasted_iota(jnp.int32, (BLK, CT), 0) + i * BLK
        ones = jnp.ones((BLK, 1), jnp.float32)
        iota = lax.broadcasted_iota(jnp.int32, (BLK, CT), 1)
        coln = lax.broadcasted_iota(jnp.int32, (1, N), 1)
        seg_lo = jnp.min(bblk)
        seg_hi = jnp.max(bblk)
        col_first = jnp.min(jnp.where(bcol == seg_lo, coln, N))
        col_last = jnp.max(jnp.where(bcol == seg_hi, coln, 0))
        t_lo = col_first // CT
        t_hi = col_last // CT + 1

        sqb = jnp.sum(fb * fb, axis=1, keepdims=True)

        def dist_tile(t, carry):
            ft = f_ref[pl.ds(t * CT, CT), :]
            mm = lax.dot_general(fb, ft, (((1,), (1,)), ((), ())),
                                 preferred_element_type=jnp.float32)
            sqt = jnp.sum(ft * ft, axis=1, keepdims=True)
            sqj = lax.dot_general(ones, sqt, (((1,), (1,)), ((), ())),
                                  preferred_element_type=jnp.float32,
                                  precision=lax.Precision.HIGHEST)
            dt = (sqb + sqj) - 2.0 * mm
            colt = iota + t * CT
            bct = bcol3_ref[t]
            invalid = (bblk != bct) | (colt == row)
            d_ref[t] = jnp.where(invalid, jnp.inf, dt)
            return carry

        lax.fori_loop(t_lo, t_hi, dist_tile, 0)

        kiota = lax.broadcasted_iota(jnp.int32, (BLK, K), 1)

        def select(k, carry):
            v, lc, acc = carry

            def scan_tile(t, c2):
                best_v, best_j = c2
                dt = d_ref[t]
                colt = iota + t * CT
                elig = (dt > v) | ((dt == v) & (colt > lc))
                dte = jnp.where(elig, dt, jnp.inf)
                mt = jnp.min(dte, axis=1, keepdims=True)
                jt = jnp.min(jnp.where(dte <= mt, colt, N), axis=1,
                             keepdims=True)
                take = (mt < best_v) | ((mt == best_v) & (jt < best_j))
                return (jnp.where(take, mt, best_v),
                        jnp.where(take, jt, best_j))

            best_v = jnp.full((BLK, 1), jnp.inf, jnp.float32)
            best_j = jnp.full((BLK, 1), N, jnp.int32)
            best_v, best_j = lax.fori_loop(t_lo, t_hi, scan_tile,
                                           (best_v, best_j))
            acc = jnp.where(kiota == k, best_j, acc)
            return best_v, best_j, acc

        v0 = jnp.full((BLK, 1), -jnp.inf, jnp.float32)
        lc0 = jnp.full((BLK, 1), -1, jnp.int32)
        acc0 = jnp.zeros((BLK, K), jnp.int32)
        _, _, acc = lax.fori_loop(0, K, select, (v0, lc0, acc0))
        idx_ref[...] = acc
        as_ref[...] = lax.dot_general(fb, wd_ref[...], (((1,), (0,)), ((), ())),
                                      preferred_element_type=jnp.float32) + c_ref[...]
        bs_ref[...] = lax.dot_general(fb, wbs_ref[...], (((1,), (0,)), ((), ())),
                                      preferred_element_type=jnp.float32)

    return pl.pallas_call(
        body,
        grid=(nblk,),
        in_specs=[
            pl.BlockSpec((BLK, din), lambda i: (i, 0)),
            pl.BlockSpec((N, din), lambda i: (0, 0)),
            pl.BlockSpec((1, N), lambda i: (0, 0)),
            pl.BlockSpec((NT, 1, CT), lambda i: (0, 0, 0)),
            pl.BlockSpec((BLK, 1), lambda i: (i, 0)),
            pl.BlockSpec((din, dout), lambda i: (0, 0)),
            pl.BlockSpec((din, dpad), lambda i: (0, 0)),
            pl.BlockSpec((1, dout), lambda i: (0, 0)),
        ],
        out_specs=[
            pl.BlockSpec((BLK, K), lambda i: (i, 0)),
            pl.BlockSpec((BLK, dout), lambda i: (i, 0)),
            pl.BlockSpec((BLK, dpad), lambda i: (i, 0)),
        ],
        out_shape=[
            jax.ShapeDtypeStruct((N, K), jnp.int32),
            jax.ShapeDtypeStruct((N, dout), jnp.float32),
            jax.ShapeDtypeStruct((N, dpad), jnp.float32),
        ],
        scratch_shapes=[pltpu.VMEM((NT, BLK, CT), jnp.float32)],
        compiler_params=pltpu.CompilerParams(
            dimension_semantics=("arbitrary",),
        ),
    )(feat, feat, bcol, bcol.reshape(NT, 1, CT), bblk, Wd, Wbs, c)


def _sc_gather_max(idx_flat, Bs, As):
    dpad = Bs.shape[1]
    dout = As.shape[1]
    half = CN * K // 2
    mesh = plsc.VectorSubcoreMesh(core_axis_name="c", subcore_axis_name="s")

    @functools.partial(
        pl.kernel,
        out_type=jax.ShapeDtypeStruct((N, dout), jnp.float32),
        mesh=mesh,
        scratch_types=[
            pltpu.VMEM((half,), jnp.int32),
            pltpu.VMEM((half,), jnp.int32),
            pltpu.VMEM((half, dpad), jnp.float32),
            pltpu.VMEM((half, dpad), jnp.float32),
            pltpu.VMEM((CN, dout), jnp.float32),
            pltpu.VMEM((CN, dout), jnp.float32),
            pltpu.SemaphoreType.DMA,
        ],
    )
    def run(idx_hbm, bs_hbm, as_hbm, out_hbm,
            idx_a, idx_b, rows_a, rows_b, as_v, out_v, sem):
        wid = lax.axis_index("s") * 2 + lax.axis_index("c")

        def chunk(ch, carry):
            nb = wid * NPW + ch * CN
            pltpu.sync_copy(idx_hbm.at[pl.ds(nb * K, half)], idx_a)
            pltpu.sync_copy(idx_hbm.at[pl.ds(nb * K + half, half)], idx_b)
            cp_a = pltpu.async_copy(bs_hbm.at[idx_a], rows_a, sem)
            cp_b = pltpu.async_copy(bs_hbm.at[idx_b], rows_b, sem)
            pltpu.sync_copy(as_hbm.at[pl.ds(nb, CN)], as_v)
            cp_a.wait()
            cp_b.wait()
            for n in range(CN):
                rows = rows_a if n < CN // 2 else rows_b
                base = (n % (CN // 2)) * K
                for jj in range(dout // 16):
                    sl = pl.ds(jj * 16, 16)
                    acc = rows[base, sl]
                    for kk in range(1, K):
                        acc = jnp.maximum(acc, rows[base + kk, sl])
                    out_v[n, sl] = jnp.maximum(acc + as_v[n, sl], 0.0)
            pltpu.sync_copy(out_v, out_hbm.at[pl.ds(nb, CN)])
            return carry

        lax.fori_loop(0, NPW // CN, chunk, 0)

    return run(idx_flat, Bs, As)


def _head(x1, x2, x3, x4, bblk, A1, c1, A2, c2, A3, c3):
    def body(x1_ref, x2_ref, x3_ref, x4_ref, b_ref,
             a1_ref, c1_ref, a2_ref, c2_ref, a3_ref, c3_ref, out_ref):
        xc = jnp.concatenate(
            [x1_ref[...], x2_ref[...], x3_ref[...], x4_ref[...]], axis=1)
        b = b_ref[...]
        rows = []
        for g in range(G):
            m = (b == g)
            rows.append(jnp.max(jnp.where(m, xc, -jnp.inf), axis=0,
                                keepdims=True))
        p = jnp.concatenate(rows, axis=0)
        h = jnp.maximum(
            lax.dot_general(p, a1_ref[...], (((1,), (0,)), ((), ())),
                            preferred_element_type=jnp.float32) + c1_ref[...],
            0.0)
        h = jnp.maximum(
            lax.dot_general(h, a2_ref[...], (((1,), (0,)), ((), ())),
                            preferred_element_type=jnp.float32) + c2_ref[...],
            0.0)
        o = lax.dot_general(h, a3_ref[...], (((1,), (0,)), ((), ())),
                            preferred_element_type=jnp.float32) + c3_ref[...]
        mx = jnp.max(o, axis=1, keepdims=True)
        e = jnp.exp(o - mx)
        lse = jnp.log(jnp.sum(e, axis=1, keepdims=True))
        out_ref[...] = o - mx - lse

    return pl.pallas_call(
        body,
        out_shape=jax.ShapeDtypeStruct((G, 40), jnp.float32),
    )(x1, x2, x3, x4, bblk, A1, c1, A2, c2, A3, c3)


def kernel(x, batch, W1, b1, g1, be1, W2, b2, g2, be2, W3, b3, g3, be3,
           W4, b4, g4, be4, Wl1, bl1, gl1, bel1, Wf1, bf1, gf1, bef1,
           Wf2, bf2):
    scale = 1.0 / jnp.sqrt(jnp.float32(1.0) + EPS)
    bcol = batch.reshape(1, N)
    bblk = batch.reshape(N, 1)

    def prep(W, b, g, be, din):
        s = g * scale
        Wa = W[:, :din]
        Wb = W[:, din:]
        Wd = ((Wa - Wb) * s[:, None]).T
        Wbs = (Wb * s[:, None]).T
        dout = Wbs.shape[1]
        if dout < 128:
            Wbs = jnp.pad(Wbs, ((0, 0), (0, 128 - dout)))
        c = (s * b + be)[None, :]
        return Wd, Wbs, c

    feats = x
    outs = []
    for (W, b, g, be, din) in ((W1, b1, g1, be1, 3), (W2, b2, g2, be2, 64),
                               (W3, b3, g3, be3, 64), (W4, b4, g4, be4, 128)):
        Wd, Wbs, c = prep(W, b, g, be, din)
        idx, As, Bs = _knn_and_mm(feats, bcol, bblk, Wd, Wbs, c)
        out = _sc_gather_max(idx.reshape(-1), Bs, As)
        outs.append(out)
        feats = out

    sl1 = gl1 * scale
    A1 = Wl1.T * sl1[None, :]
    c1 = (sl1 * bl1 + bel1)[None, :]
    sf1 = gf1 * scale
    A2 = Wf1.T * sf1[None, :]
    c2 = (sf1 * bf1 + bef1)[None, :]
    A3 = Wf2.T
    c3 = bf2[None, :]
    return _head(outs[0], outs[1], outs[2], outs[3], bblk,
                 A1, c1, A2, c2, A3, c3)


# speedup vs baseline: 1.1967x; 1.1967x over previous
import functools

import jax
import jax.numpy as jnp
from jax import lax
from jax.experimental import pallas as pl
from jax.experimental.pallas import tpu as pltpu
from jax.experimental.pallas import tpu_sc as plsc

N = 4096
K = 20
G = 4
EPS = 1e-5
BLK = 512
NW = 32
NPW = N // NW
CN = 8


def _knn_and_mm(feat, bcol, bblk, Wd, Wbs, c):
    din = feat.shape[1]
    dout = Wd.shape[1]
    dpad = Wbs.shape[1]
    nblk = N // BLK

    CT = 512
    NT = N // CT

    def body(fb_ref, f_ref, bcol_ref, bcol3_ref, bblk_ref, wd_ref, wbs_ref,
             c_ref, idx_ref, as_ref, bs_ref, d_ref):
        i = pl.program_id(0)
        fb = fb_ref[...]
        bblk = bblk_ref[...]
        bcol = bcol_ref[...]
        row = lax.broadcasted_iota(jnp.int32, (BLK, CT), 0) + i * BLK
        ones = jnp.ones((BLK, 1), jnp.float32)
        iota = lax.broadcasted_iota(jnp.int32, (BLK, CT), 1)
        coln = lax.broadcasted_iota(jnp.int32, (1, N), 1)
        seg_lo = jnp.min(bblk)
        seg_hi = jnp.max(bblk)
        col_first = jnp.min(jnp.where(bcol == seg_lo, coln, N))
        col_last = jnp.max(jnp.where(bcol == seg_hi, coln, 0))
        t_lo = col_first // CT
        t_hi = col_last // CT + 1

        sqb = jnp.sum(fb * fb, axis=1, keepdims=True)

        def dist_tile(t, carry):
            ft = f_ref[pl.ds(t * CT, CT), :]
            mm = lax.dot_general(fb, ft, (((1,), (1,)), ((), ())),
                                 preferred_element_type=jnp.float32)
            sqt = jnp.sum(ft * ft, axis=1, keepdims=True)
            sqj = lax.dot_general(ones, sqt, (((1,), (1,)), ((), ())),
                                  preferred_element_type=jnp.float32,
                                  precision=lax.Precision.HIGHEST)
            dt = (sqb + sqj) - 2.0 * mm
            colt = iota + t * CT
            bct = bcol3_ref[t]
            invalid = (bblk != bct) | (colt == row)
            d_ref[t] = jnp.where(invalid, jnp.inf, dt)
            return carry

        lax.fori_loop(t_lo, t_hi, dist_tile, 0)

        kiota = lax.broadcasted_iota(jnp.int32, (BLK, K), 1)

        def select(k, carry):
            v, lc, acc = carry

            def scan_tile(t, c2):
                best_v, best_j = c2
                dt = d_ref[t]
                colt = iota + t * CT
                elig = (dt > v) | ((dt == v) & (colt > lc))
                dte = jnp.where(elig, dt, jnp.inf)
                mt = jnp.min(dte, axis=1, keepdims=True)
                jt = jnp.min(jnp.where(dte <= mt, colt, N), axis=1,
                             keepdims=True)
                take = (mt < best_v) | ((mt == best_v) & (jt < best_j))
                return (jnp.where(take, mt, best_v),
                        jnp.where(take, jt, best_j))

            best_v = jnp.full((BLK, 1), jnp.inf, jnp.float32)
            best_j = jnp.full((BLK, 1), N, jnp.int32)
            best_v, best_j = lax.fori_loop(t_lo, t_hi, scan_tile,
                                           (best_v, best_j))
            acc = jnp.where(kiota == k, best_j, acc)
            return best_v, best_j, acc

        v0 = jnp.full((BLK, 1), -jnp.inf, jnp.float32)
        lc0 = jnp.full((BLK, 1), -1, jnp.int32)
        acc0 = jnp.zeros((BLK, K), jnp.int32)
        _, _, acc = lax.fori_loop(0, K, select, (v0, lc0, acc0))
        idx_ref[...] = acc
        as_ref[...] = lax.dot_general(fb, wd_ref[...], (((1,), (0,)), ((), ())),
                                      preferred_element_type=jnp.float32) + c_ref[...]
        bs_ref[...] = lax.dot_general(fb, wbs_ref[...], (((1,), (0,)), ((), ())),
                                      preferred_element_type=jnp.float32)

    return pl.pallas_call(
        body,
        grid=(nblk,),
        in_specs=[
            pl.BlockSpec((BLK, din), lambda i: (i, 0)),
            pl.BlockSpec((N, din), lambda i: (0, 0)),
            pl.BlockSpec((1, N), lambda i: (0, 0)),
            pl.BlockSpec((NT, 1, CT), lambda i: (0, 0, 0)),
            pl.BlockSpec((BLK, 1), lambda i: (i, 0)),
            pl.BlockSpec((din, dout), lambda i: (0, 0)),
            pl.BlockSpec((din, dpad), lambda i: (0, 0)),
            pl.BlockSpec((1, dout), lambda i: (0, 0)),
        ],
        out_specs=[
            pl.BlockSpec((BLK, K), lambda i: (i, 0)),
            pl.BlockSpec((BLK, dout), lambda i: (i, 0)),
            pl.BlockSpec((BLK, dpad), lambda i: (i, 0)),
        ],
        out_shape=[
            jax.ShapeDtypeStruct((N, K), jnp.int32),
            jax.ShapeDtypeStruct((N, dout), jnp.float32),
            jax.ShapeDtypeStruct((N, dpad), jnp.float32),
        ],
        scratch_shapes=[pltpu.VMEM((NT, BLK, CT), jnp.float32)],
        compiler_params=pltpu.CompilerParams(
            dimension_semantics=("arbitrary",),
        ),
    )(feat, feat, bcol, bcol.reshape(NT, 1, CT), bblk, Wd, Wbs, c)


def _sc_gather_max(idx_flat, Bs, As):
    dpad = Bs.shape[1]
    dout = As.shape[1]
    half = CN * K // 2
    NCH = NPW // CN
    mesh = plsc.VectorSubcoreMesh(core_axis_name="c", subcore_axis_name="s")

    @functools.partial(
        pl.kernel,
        out_type=jax.ShapeDtypeStruct((N, dout), jnp.float32),
        mesh=mesh,
        scratch_types=[
            pltpu.VMEM((NPW * K,), jnp.int32),
            pltpu.VMEM((half,), jnp.int32),
            pltpu.VMEM((half,), jnp.int32),
            pltpu.VMEM((half,), jnp.int32),
            pltpu.VMEM((half,), jnp.int32),
            pltpu.VMEM((half, dpad), jnp.float32),
            pltpu.VMEM((half, dpad), jnp.float32),
            pltpu.VMEM((half, dpad), jnp.float32),
            pltpu.VMEM((half, dpad), jnp.float32),
            pltpu.VMEM((CN, dout), jnp.float32),
            pltpu.VMEM((CN, dout), jnp.float32),
            pltpu.VMEM((NPW, dout), jnp.float32),
            pltpu.SemaphoreType.DMA,
            pltpu.SemaphoreType.DMA,
        ],
    )
    def run(idx_hbm, bs_hbm, as_hbm, out_hbm, idx_v,
            ia1, ia2, ib1, ib2, ra1, ra2, rb1, rb2,
            aa, ab, out_v, sem_a, sem_b):
        wid = lax.axis_index("s") * 2 + lax.axis_index("c")
        base = wid * NPW
        pltpu.sync_copy(idx_hbm.at[pl.ds(base * K, NPW * K)], idx_v)

        def stage_idx(c, i1, i2):
            for h in range(2 * half // 16):
                dst = (i1, i2)[h // (half // 16)]
                dst[pl.ds((h % (half // 16)) * 16, 16)] = (
                    idx_v[pl.ds(c * CN * K + h * 16, 16)])

        def fire(c, i1, i2, r1, r2, av, sem):
            stage_idx(c, i1, i2)
            pltpu.async_copy(bs_hbm.at[i1], r1, sem)
            pltpu.async_copy(bs_hbm.at[i2], r2, sem)
            pltpu.async_copy(as_hbm.at[pl.ds(base + c * CN, CN)], av, sem)

        def drain(i1, i2, r1, r2, av, sem):
            pltpu.make_async_copy(bs_hbm.at[i1], r1, sem).wait()
            pltpu.make_async_copy(bs_hbm.at[i2], r2, sem).wait()
            pltpu.make_async_copy(as_hbm.at[pl.ds(base, CN)], av, sem).wait()

        def compute(c, r1, r2, av):
            def make_nbody(rows, off):
                def nbody(n, carry):
                    for jj in range(dout // 16):
                        sl = pl.ds(jj * 16, 16)
                        acc = rows[n * K, sl]
                        for kk in range(1, K):
                            acc = jnp.maximum(acc, rows[n * K + kk, sl])
                        out_v[c * CN + off + n, sl] = jnp.maximum(
                            acc + av[off + n, sl], 0.0)
                    return carry
                return nbody

            lax.fori_loop(0, CN // 2, make_nbody(r1, 0), 0)
            lax.fori_loop(0, CN // 2, make_nbody(r2, CN // 2), 0)

        fire(0, ia1, ia2, ra1, ra2, aa, sem_a)

        def pair(i, carry):
            c0 = 2 * i
            fire(c0 + 1, ib1, ib2, rb1, rb2, ab, sem_b)
            drain(ia1, ia2, ra1, ra2, aa, sem_a)
            compute(c0, ra1, ra2, aa)

            @pl.when(i < NCH // 2 - 1)
            def _():
                fire(c0 + 2, ia1, ia2, ra1, ra2, aa, sem_a)

            drain(ib1, ib2, rb1, rb2, ab, sem_b)
            compute(c0 + 1, rb1, rb2, ab)
            return carry

        lax.fori_loop(0, NCH // 2, pair, 0)
        pltpu.sync_copy(out_v, out_hbm.at[pl.ds(base, NPW)])

    return run(idx_flat, Bs, As)


def _head(x1, x2, x3, x4, bblk, A1, c1, A2, c2, A3, c3):
    def body(x1_ref, x2_ref, x3_ref, x4_ref, b_ref,
             a1_ref, c1_ref, a2_ref, c2_ref, a3_ref, c3_ref, out_ref):
        xc = jnp.concatenate(
            [x1_ref[...], x2_ref[...], x3_ref[...], x4_ref[...]], axis=1)
        b = b_ref[...]
        rows = []
        for g in range(G):
            m = (b == g)
            rows.append(jnp.max(jnp.where(m, xc, -jnp.inf), axis=0,
                                keepdims=True))
        p = jnp.concatenate(rows, axis=0)
        h = jnp.maximum(
            lax.dot_general(p, a1_ref[...], (((1,), (0,)), ((), ())),
                            preferred_element_type=jnp.float32) + c1_ref[...],
            0.0)
        h = jnp.maximum(
            lax.dot_general(h, a2_ref[...], (((1,), (0,)), ((), ())),
                            preferred_element_type=jnp.float32) + c2_ref[...],
            0.0)
        o = lax.dot_general(h, a3_ref[...], (((1,), (0,)), ((), ())),
                            preferred_element_type=jnp.float32) + c3_ref[...]
        mx = jnp.max(o, axis=1, keepdims=True)
        e = jnp.exp(o - mx)
        lse = jnp.log(jnp.sum(e, axis=1, keepdims=True))
        out_ref[...] = o - mx - lse

    return pl.pallas_call(
        body,
        out_shape=jax.ShapeDtypeStruct((G, 40), jnp.float32),
    )(x1, x2, x3, x4, bblk, A1, c1, A2, c2, A3, c3)


def kernel(x, batch, W1, b1, g1, be1, W2, b2, g2, be2, W3, b3, g3, be3,
           W4, b4, g4, be4, Wl1, bl1, gl1, bel1, Wf1, bf1, gf1, bef1,
           Wf2, bf2):
    scale = 1.0 / jnp.sqrt(jnp.float32(1.0) + EPS)
    bcol = batch.reshape(1, N)
    bblk = batch.reshape(N, 1)

    def prep(W, b, g, be, din):
        s = g * scale
        Wa = W[:, :din]
        Wb = W[:, din:]
        Wd = ((Wa - Wb) * s[:, None]).T
        Wbs = (Wb * s[:, None]).T
        dout = Wbs.shape[1]
        if dout < 128:
            Wbs = jnp.pad(Wbs, ((0, 0), (0, 128 - dout)))
        c = (s * b + be)[None, :]
        return Wd, Wbs, c

    feats = x
    outs = []
    for (W, b, g, be, din) in ((W1, b1, g1, be1, 3), (W2, b2, g2, be2, 64),
                               (W3, b3, g3, be3, 64), (W4, b4, g4, be4, 128)):
        Wd, Wbs, c = prep(W, b, g, be, din)
        idx, As, Bs = _knn_and_mm(feats, bcol, bblk, Wd, Wbs, c)
        out = _sc_gather_max(idx.reshape(-1), Bs, As)
        outs.append(out)
        feats = out

    sl1 = gl1 * scale
    A1 = Wl1.T * sl1[None, :]
    c1 = (sl1 * bl1 + bel1)[None, :]
    sf1 = gf1 * scale
    A2 = Wf1.T * sf1[None, :]
    c2 = (sf1 * bf1 + bef1)[None, :]
    A3 = Wf2.T
    c3 = bf2[None, :]
    return _head(outs[0], outs[1], outs[2], outs[3], bblk,
                 A1, c1, A2, c2, A3, c3)


# speedup vs baseline: 1.4315x; 1.1962x over previous
import functools

import jax
import jax.numpy as jnp
from jax import lax
from jax.experimental import pallas as pl
from jax.experimental.pallas import tpu as pltpu
from jax.experimental.pallas import tpu_sc as plsc

N = 4096
K = 20
G = 4
EPS = 1e-5
BLK = 512
NW = 32
NPW = N // NW
CN = 8


def _knn_and_mm(feat, bcol, bblk, Wd, Wbs, c):
    din = feat.shape[1]
    dout = Wd.shape[1]
    dpad = Wbs.shape[1]
    nblk = N // BLK

    CT = 512
    NT = N // CT

    def body(fb_ref, f_ref, bcol_ref, bcol3_ref, bblk_ref, wd_ref, wbs_ref,
             c_ref, idx_ref, as_ref, bs_ref, d_ref):
        i = pl.program_id(0)
        fb = fb_ref[...]
        bblk = bblk_ref[...]
        bcol = bcol_ref[...]
        row = lax.broadcasted_iota(jnp.int32, (BLK, CT), 0) + i * BLK
        ones = jnp.ones((BLK, 1), jnp.float32)
        iota = lax.broadcasted_iota(jnp.int32, (BLK, CT), 1)
        coln = lax.broadcasted_iota(jnp.int32, (1, N), 1)
        seg_lo = jnp.min(bblk)
        seg_hi = jnp.max(bblk)
        col_first = jnp.min(jnp.where(bcol == seg_lo, coln, N))
        col_last = jnp.max(jnp.where(bcol == seg_hi, coln, 0))
        t_lo = col_first // CT
        t_hi = col_last // CT + 1

        sqb = jnp.sum(fb * fb, axis=1, keepdims=True)

        def dist_tile(t, carry):
            ft = f_ref[pl.ds(t * CT, CT), :]
            mm = lax.dot_general(fb, ft, (((1,), (1,)), ((), ())),
                                 preferred_element_type=jnp.float32)
            sqt = jnp.sum(ft * ft, axis=1, keepdims=True)
            sqj = lax.dot_general(ones, sqt, (((1,), (1,)), ((), ())),
                                  preferred_element_type=jnp.float32,
                                  precision=lax.Precision.HIGHEST)
            dt = (sqb + sqj) - 2.0 * mm
            colt = iota + t * CT
            bct = bcol3_ref[t]
            invalid = (bblk != bct) | (colt == row)
            d_ref[t] = jnp.where(invalid, jnp.inf, dt)
            return carry

        lax.fori_loop(t_lo, t_hi, dist_tile, 0)

        kiota = lax.broadcasted_iota(jnp.int32, (BLK, K), 1)

        def select(k, carry):
            lc, acc = carry

            def scan_tile(t, c2):
                best_v, best_j = c2
                colt = iota + t * CT
                dt = jnp.where(colt == lc, jnp.inf, d_ref[t])
                d_ref[t] = dt
                mt = jnp.min(dt, axis=1, keepdims=True)
                jt = jnp.min(jnp.where(dt <= mt, colt, N), axis=1,
                             keepdims=True)
                take = (mt < best_v) | ((mt == best_v) & (jt < best_j))
                return (jnp.where(take, mt, best_v),
                        jnp.where(take, jt, best_j))

            best_v = jnp.full((BLK, 1), jnp.inf, jnp.float32)
            best_j = jnp.full((BLK, 1), N, jnp.int32)
            best_v, best_j = lax.fori_loop(t_lo, t_hi, scan_tile,
                                           (best_v, best_j))
            acc = jnp.where(kiota == k, best_j, acc)
            return best_j, acc

        lc0 = jnp.full((BLK, 1), -1, jnp.int32)
        acc0 = jnp.zeros((BLK, K), jnp.int32)
        _, acc = lax.fori_loop(0, K, select, (lc0, acc0))
        idx_ref[...] = acc
        as_ref[...] = lax.dot_general(fb, wd_ref[...], (((1,), (0,)), ((), ())),
                                      preferred_element_type=jnp.float32) + c_ref[...]
        bs_ref[...] = lax.dot_general(fb, wbs_ref[...], (((1,), (0,)), ((), ())),
                                      preferred_element_type=jnp.float32)

    return pl.pallas_call(
        body,
        grid=(nblk,),
        in_specs=[
            pl.BlockSpec((BLK, din), lambda i: (i, 0)),
            pl.BlockSpec((N, din), lambda i: (0, 0)),
            pl.BlockSpec((1, N), lambda i: (0, 0)),
            pl.BlockSpec((NT, 1, CT), lambda i: (0, 0, 0)),
            pl.BlockSpec((BLK, 1), lambda i: (i, 0)),
            pl.BlockSpec((din, dout), lambda i: (0, 0)),
            pl.BlockSpec((din, dpad), lambda i: (0, 0)),
            pl.BlockSpec((1, dout), lambda i: (0, 0)),
        ],
        out_specs=[
            pl.BlockSpec((BLK, K), lambda i: (i, 0)),
            pl.BlockSpec((BLK, dout), lambda i: (i, 0)),
            pl.BlockSpec((BLK, dpad), lambda i: (i, 0)),
        ],
        out_shape=[
            jax.ShapeDtypeStruct((N, K), jnp.int32),
            jax.ShapeDtypeStruct((N, dout), jnp.float32),
            jax.ShapeDtypeStruct((N, dpad), jnp.float32),
        ],
        scratch_shapes=[pltpu.VMEM((NT, BLK, CT), jnp.float32)],
        compiler_params=pltpu.CompilerParams(
            dimension_semantics=("arbitrary",),
        ),
    )(feat, feat, bcol, bcol.reshape(NT, 1, CT), bblk, Wd, Wbs, c)


def _sc_gather_max(idx_flat, Bs, As):
    dpad = Bs.shape[1]
    dout = As.shape[1]
    half = CN * K // 2
    NCH = NPW // CN
    mesh = plsc.VectorSubcoreMesh(core_axis_name="c", subcore_axis_name="s")

    @functools.partial(
        pl.kernel,
        out_type=jax.ShapeDtypeStruct((N, dout), jnp.float32),
        mesh=mesh,
        scratch_types=[
            pltpu.VMEM((NPW * K,), jnp.int32),
            pltpu.VMEM((half,), jnp.int32),
            pltpu.VMEM((half,), jnp.int32),
            pltpu.VMEM((half,), jnp.int32),
            pltpu.VMEM((half,), jnp.int32),
            pltpu.VMEM((half, dpad), jnp.float32),
            pltpu.VMEM((half, dpad), jnp.float32),
            pltpu.VMEM((half, dpad), jnp.float32),
            pltpu.VMEM((half, dpad), jnp.float32),
            pltpu.VMEM((CN, dout), jnp.float32),
            pltpu.VMEM((CN, dout), jnp.float32),
            pltpu.VMEM((NPW, dout), jnp.float32),
            pltpu.SemaphoreType.DMA,
            pltpu.SemaphoreType.DMA,
        ],
    )
    def run(idx_hbm, bs_hbm, as_hbm, out_hbm, idx_v,
            ia1, ia2, ib1, ib2, ra1, ra2, rb1, rb2,
            aa, ab, out_v, sem_a, sem_b):
        wid = lax.axis_index("s") * 2 + lax.axis_index("c")
        base = wid * NPW
        pltpu.sync_copy(idx_hbm.at[pl.ds(base * K, NPW * K)], idx_v)

        def stage_idx(c, i1, i2):
            for h in range(2 * half // 16):
                dst = (i1, i2)[h // (half // 16)]
                dst[pl.ds((h % (half // 16)) * 16, 16)] = (
                    idx_v[pl.ds(c * CN * K + h * 16, 16)])

        def fire(c, i1, i2, r1, r2, av, sem):
            stage_idx(c, i1, i2)
            pltpu.async_copy(bs_hbm.at[i1], r1, sem)
            pltpu.async_copy(bs_hbm.at[i2], r2, sem)
            pltpu.async_copy(as_hbm.at[pl.ds(base + c * CN, CN)], av, sem)

        def drain(i1, i2, r1, r2, av, sem):
            pltpu.make_async_copy(bs_hbm.at[i1], r1, sem).wait()
            pltpu.make_async_copy(bs_hbm.at[i2], r2, sem).wait()
            pltpu.make_async_copy(as_hbm.at[pl.ds(base, CN)], av, sem).wait()

        def compute(c, r1, r2, av):
            def make_nbody(rows, off):
                def nbody(n, carry):
                    for jj in range(dout // 16):
                        sl = pl.ds(jj * 16, 16)
                        acc = rows[n * K, sl]
                        for kk in range(1, K):
                            acc = jnp.maximum(acc, rows[n * K + kk, sl])
                        out_v[c * CN + off + n, sl] = jnp.maximum(
                            acc + av[off + n, sl], 0.0)
                    return carry
                return nbody

            lax.fori_loop(0, CN // 2, make_nbody(r1, 0), 0)
            lax.fori_loop(0, CN // 2, make_nbody(r2, CN // 2), 0)

        fire(0, ia1, ia2, ra1, ra2, aa, sem_a)

        def pair(i, carry):
            c0 = 2 * i
            fire(c0 + 1, ib1, ib2, rb1, rb2, ab, sem_b)
            drain(ia1, ia2, ra1, ra2, aa, sem_a)
            compute(c0, ra1, ra2, aa)

            @pl.when(i < NCH // 2 - 1)
            def _():
                fire(c0 + 2, ia1, ia2, ra1, ra2, aa, sem_a)

            drain(ib1, ib2, rb1, rb2, ab, sem_b)
            compute(c0 + 1, rb1, rb2, ab)
            return carry

        lax.fori_loop(0, NCH // 2, pair, 0)
        pltpu.sync_copy(out_v, out_hbm.at[pl.ds(base, NPW)])

    return run(idx_flat, Bs, As)


def _head(x1, x2, x3, x4, bblk, A1, c1, A2, c2, A3, c3):
    def body(x1_ref, x2_ref, x3_ref, x4_ref, b_ref,
             a1_ref, c1_ref, a2_ref, c2_ref, a3_ref, c3_ref, out_ref):
        xc = jnp.concatenate(
            [x1_ref[...], x2_ref[...], x3_ref[...], x4_ref[...]], axis=1)
        b = b_ref[...]
        rows = []
        for g in range(G):
            m = (b == g)
            rows.append(jnp.max(jnp.where(m, xc, -jnp.inf), axis=0,
                                keepdims=True))
        p = jnp.concatenate(rows, axis=0)
        h = jnp.maximum(
            lax.dot_general(p, a1_ref[...], (((1,), (0,)), ((), ())),
                            preferred_element_type=jnp.float32) + c1_ref[...],
            0.0)
        h = jnp.maximum(
            lax.dot_general(h, a2_ref[...], (((1,), (0,)), ((), ())),
                            preferred_element_type=jnp.float32) + c2_ref[...],
            0.0)
        o = lax.dot_general(h, a3_ref[...], (((1,), (0,)), ((), ())),
                            preferred_element_type=jnp.float32) + c3_ref[...]
        mx = jnp.max(o, axis=1, keepdims=True)
        e = jnp.exp(o - mx)
        lse = jnp.log(jnp.sum(e, axis=1, keepdims=True))
        out_ref[...] = o - mx - lse

    return pl.pallas_call(
        body,
        out_shape=jax.ShapeDtypeStruct((G, 40), jnp.float32),
    )(x1, x2, x3, x4, bblk, A1, c1, A2, c2, A3, c3)


def kernel(x, batch, W1, b1, g1, be1, W2, b2, g2, be2, W3, b3, g3, be3,
           W4, b4, g4, be4, Wl1, bl1, gl1, bel1, Wf1, bf1, gf1, bef1,
           Wf2, bf2):
    scale = 1.0 / jnp.sqrt(jnp.float32(1.0) + EPS)
    bcol = batch.reshape(1, N)
    bblk = batch.reshape(N, 1)

    def prep(W, b, g, be, din):
        s = g * scale
        Wa = W[:, :din]
        Wb = W[:, din:]
        Wd = ((Wa - Wb) * s[:, None]).T
        Wbs = (Wb * s[:, None]).T
        dout = Wbs.shape[1]
        if dout < 128:
            Wbs = jnp.pad(Wbs, ((0, 0), (0, 128 - dout)))
        c = (s * b + be)[None, :]
        return Wd, Wbs, c

    feats = x
    outs = []
    for (W, b, g, be, din) in ((W1, b1, g1, be1, 3), (W2, b2, g2, be2, 64),
                               (W3, b3, g3, be3, 64), (W4, b4, g4, be4, 128)):
        Wd, Wbs, c = prep(W, b, g, be, din)
        idx, As, Bs = _knn_and_mm(feats, bcol, bblk, Wd, Wbs, c)
        out = _sc_gather_max(idx.reshape(-1), Bs, As)
        outs.append(out)
        feats = out

    sl1 = gl1 * scale
    A1 = Wl1.T * sl1[None, :]
    c1 = (sl1 * bl1 + bel1)[None, :]
    sf1 = gf1 * scale
    A2 = Wf1.T * sf1[None, :]
    c2 = (sf1 * bf1 + bef1)[None, :]
    A3 = Wf2.T
    c3 = bf2[None, :]
    return _head(outs[0], outs[1], outs[2], outs[3], bblk,
                 A1, c1, A2, c2, A3, c3)
